# baseline (speedup 1.0000x reference)
import jax
import jax.numpy as jnp
from jax.experimental import pallas as pl
from jax.experimental.pallas import tpu as pltpu


def _gcn_body(x_ref, w_ref, b_ref, adj_ref, out_ref, support_ref):
    i = pl.program_id(0)
    k = pl.program_id(1)
    n = support_ref.shape[0]
    bk = adj_ref.shape[1]

    @pl.when((i == 0) & (k == 0))
    def _():
        support_ref[...] = jnp.dot(x_ref[...], w_ref[...],
                                   preferred_element_type=jnp.float32)

    @pl.when(k == 0)
    def _():
        acc = jnp.dot(adj_ref[...], support_ref[:bk, :],
                      preferred_element_type=jnp.float32)
        out_ref[...] = acc + b_ref[...]

    @pl.when(k == 1)
    def _():
        rem = n - bk
        acc = jnp.dot(adj_ref[:, :rem], support_ref[bk:, :],
                      preferred_element_type=jnp.float32)
        out_ref[...] += acc


def kernel(input, adj, W, b):
    N, d_in = input.shape
    d_out = W.shape[1]
    BM = N // 10
    BK = (N // 2 + 127) // 128 * 128

    b2 = b.reshape(1, d_out).astype(jnp.float32)

    return pl.pallas_call(
        _gcn_body,
        grid=(N // BM, 2),
        in_specs=[
            pl.BlockSpec((N, d_in), lambda i, k: (0, 0)),
            pl.BlockSpec((d_in, d_out), lambda i, k: (0, 0)),
            pl.BlockSpec((1, d_out), lambda i, k: (0, 0)),
            pl.BlockSpec((BM, BK), lambda i, k: (i, k)),
        ],
        out_specs=pl.BlockSpec((BM, d_out), lambda i, k: (i, 0)),
        out_shape=jax.ShapeDtypeStruct((N, d_out), jnp.float32),
        scratch_shapes=[pltpu.VMEM((N, d_out), jnp.float32)],
    )(input.astype(jnp.float32), W.astype(jnp.float32), b2, adj.astype(jnp.float32))
